# baseline (speedup 1.0000x reference)
import functools

import jax
import jax.numpy as jnp
from jax import lax
from jax.experimental import pallas as pl
from jax.experimental.pallas import tpu as pltpu
from jax.experimental.pallas import tpu_sc as plsc

BT = 128
NC, NS = 2, 16
NW = NC * NS


def _sc_row_gather(table, idx, out_rows, chunk_rows=64):
    D = table.shape[1]
    b_per_w = out_rows // NW
    n_chunks = b_per_w // chunk_rows
    mesh = plsc.VectorSubcoreMesh(core_axis_name="c", subcore_axis_name="s")

    @functools.partial(
        pl.kernel,
        mesh=mesh,
        out_type=jax.ShapeDtypeStruct((out_rows, D), jnp.float32),
        scratch_types=[
            pltpu.VMEM((chunk_rows,), jnp.int32),
            pltpu.VMEM((chunk_rows, D), jnp.float32),
            pltpu.SemaphoreType.DMA,
        ],
    )
    def gather_k(table_hbm, idx_hbm, out_hbm, idx_v, rows_v, sem):
        wid = lax.axis_index("s") * NC + lax.axis_index("c")
        base = wid * b_per_w
        for c in range(n_chunks):
            off = base + c * chunk_rows
            pltpu.sync_copy(idx_hbm.at[pl.ds(off, chunk_rows)], idx_v)
            pltpu.async_copy(table_hbm.at[idx_v], rows_v, sem).wait()
            pltpu.sync_copy(rows_v, out_hbm.at[pl.ds(off, chunk_rows)])

    return gather_k(table, idx)


def _moe_mlp_body(be_ref, bv_ref, x_ref, gu_ref, dn_ref, w_ref, o_ref):
    b = pl.program_id(0)
    inter = dn_ref.shape[2]

    @pl.when(bv_ref[b] == 1)
    def _():
        x = x_ref[...]
        gu = gu_ref[0]
        acc = lax.dot_general(x, gu, (((1,), (1,)), ((), ())),
                              preferred_element_type=jnp.float32)
        g = acc[:, :inter]
        u = acc[:, inter:]
        h = (g * jax.nn.sigmoid(g)) * u
        dn = dn_ref[0]
        y = lax.dot_general(h, dn, (((1,), (1,)), ((), ())),
                            preferred_element_type=jnp.float32)
        o_ref[...] = y * w_ref[...]


def _grouped_mlp(x_pad, gate_up_proj, down_proj, w_slot, block_expert,
                 block_valid):
    T_pad, H = x_pad.shape
    E, two_i, _ = gate_up_proj.shape
    inter = two_i // 2
    nb = T_pad // BT

    grid_spec = pltpu.PrefetchScalarGridSpec(
        num_scalar_prefetch=2,
        grid=(nb,),
        in_specs=[
            pl.BlockSpec((BT, H), lambda b, be, bv: (b, 0)),
            pl.BlockSpec((1, two_i, H), lambda b, be, bv: (be[b], 0, 0)),
            pl.BlockSpec((1, H, inter), lambda b, be, bv: (be[b], 0, 0)),
            pl.BlockSpec((BT, 1), lambda b, be, bv: (b, 0)),
        ],
        out_specs=pl.BlockSpec((BT, H), lambda b, be, bv: (b, 0)),
    )
    return pl.pallas_call(
        _moe_mlp_body,
        grid_spec=grid_spec,
        out_shape=jax.ShapeDtypeStruct((T_pad, H), jnp.float32),
    )(block_expert, block_valid, x_pad, gate_up_proj, down_proj, w_slot)


def _routing_metadata(top_k_index, top_k_weights, num_experts, t_pad):
    T = top_k_index.shape[0]
    e = top_k_index[:, 0].astype(jnp.int32)
    onehot = (e[:, None] == jnp.arange(num_experts, dtype=jnp.int32)[None, :])
    occ = jnp.cumsum(onehot.astype(jnp.int32), axis=0)
    rank = jnp.take_along_axis(occ, e[:, None], axis=1)[:, 0] - 1
    counts = occ[-1]
    aligned = ((counts + BT - 1) // BT) * BT
    ends = jnp.cumsum(aligned)
    starts = ends - aligned
    total_used = ends[-1]

    slot_of_token = starts[e] + rank
    tok_ids = jnp.arange(T, dtype=jnp.int32)
    gather_idx = jnp.zeros((t_pad,), jnp.int32).at[slot_of_token].set(tok_ids)
    w_slot = jnp.zeros((t_pad, 1), jnp.float32).at[slot_of_token, 0].set(
        top_k_weights[:, 0])

    nb = t_pad // BT
    bstarts = jnp.arange(nb, dtype=jnp.int32) * BT
    owner = jnp.minimum(jnp.searchsorted(ends, bstarts, side="right"),
                        num_experts - 1).astype(jnp.int32)
    valid = (bstarts < total_used).astype(jnp.int32)
    last_owner = jnp.minimum(
        jnp.searchsorted(ends, total_used - 1, side="right"),
        num_experts - 1).astype(jnp.int32)
    block_expert = jnp.where(valid == 1, owner, last_owner)
    return gather_idx, w_slot, slot_of_token, block_expert, valid


def kernel(hidden_states, top_k_index, top_k_weights, gate_up_proj, down_proj):
    T, H = hidden_states.shape
    E = gate_up_proj.shape[0]
    t_pad = T + E * BT

    gather_idx, w_slot, slot_of_token, block_expert, block_valid = (
        _routing_metadata(top_k_index, top_k_weights, E, t_pad))

    x_pad = _sc_row_gather(hidden_states, gather_idx, t_pad)
    y_pad = _grouped_mlp(x_pad, gate_up_proj, down_proj, w_slot,
                         block_expert, block_valid)
    return _sc_row_gather(y_pad, slot_of_token, T)


# speedup vs baseline: 2.0552x; 2.0552x over previous
import functools

import jax
import jax.numpy as jnp
from jax import lax
from jax.experimental import pallas as pl
from jax.experimental.pallas import tpu as pltpu
from jax.experimental.pallas import tpu_sc as plsc

BT = 128
NC, NS = 2, 16
NW = NC * NS


def _sc_row_gather(table, idx, out_rows, chunk_rows=64):
    D = table.shape[1]
    b_per_w = out_rows // NW
    n_chunks = b_per_w // chunk_rows
    mesh = plsc.VectorSubcoreMesh(core_axis_name="c", subcore_axis_name="s")

    @functools.partial(
        pl.kernel,
        mesh=mesh,
        out_type=jax.ShapeDtypeStruct((out_rows, D), jnp.float32),
        scratch_types=[
            pltpu.VMEM((chunk_rows,), jnp.int32),
            pltpu.VMEM((chunk_rows, D), jnp.float32),
            pltpu.SemaphoreType.DMA,
        ],
    )
    def gather_k(table_hbm, idx_hbm, out_hbm, idx_v, rows_v, sem):
        wid = lax.axis_index("s") * NC + lax.axis_index("c")
        base = wid * b_per_w
        for c in range(n_chunks):
            off = base + c * chunk_rows
            pltpu.sync_copy(idx_hbm.at[pl.ds(off, chunk_rows)], idx_v)
            pltpu.async_copy(table_hbm.at[idx_v], rows_v, sem).wait()
            pltpu.sync_copy(rows_v, out_hbm.at[pl.ds(off, chunk_rows)])

    return gather_k(table, idx)


def _moe_mlp_body(be_ref, bv_ref, x_ref, gu_ref, dn_ref, w_ref, o_ref):
    b = pl.program_id(0)
    inter = dn_ref.shape[2]

    @pl.when(bv_ref[b] == 1)
    def _():
        x = x_ref[...]
        gu = gu_ref[0]
        acc = lax.dot_general(x, gu, (((1,), (1,)), ((), ())),
                              preferred_element_type=jnp.float32)
        g = acc[:, :inter]
        u = acc[:, inter:]
        h = (g * jax.nn.sigmoid(g)) * u
        dn = dn_ref[0]
        y = lax.dot_general(h, dn, (((1,), (1,)), ((), ())),
                            preferred_element_type=jnp.float32)
        o_ref[...] = y * w_ref[...]


def _grouped_mlp(x_pad, gate_up_proj, down_proj, w_slot, block_expert,
                 block_valid):
    T_pad, H = x_pad.shape
    E, two_i, _ = gate_up_proj.shape
    inter = two_i // 2
    nb = T_pad // BT

    grid_spec = pltpu.PrefetchScalarGridSpec(
        num_scalar_prefetch=2,
        grid=(nb,),
        in_specs=[
            pl.BlockSpec((BT, H), lambda b, be, bv: (b, 0)),
            pl.BlockSpec((1, two_i, H), lambda b, be, bv: (be[b], 0, 0)),
            pl.BlockSpec((1, H, inter), lambda b, be, bv: (be[b], 0, 0)),
            pl.BlockSpec((BT, 1), lambda b, be, bv: (b, 0)),
        ],
        out_specs=pl.BlockSpec((BT, H), lambda b, be, bv: (b, 0)),
    )
    return pl.pallas_call(
        _moe_mlp_body,
        grid_spec=grid_spec,
        out_shape=jax.ShapeDtypeStruct((T_pad, H), jnp.float32),
    )(block_expert, block_valid, x_pad, gate_up_proj, down_proj, w_slot)


def _routing_metadata(top_k_index, top_k_weights, num_experts, t_pad):
    T = top_k_index.shape[0]
    e = top_k_index[:, 0].astype(jnp.int32)
    eids = jnp.arange(num_experts, dtype=jnp.int32)
    onehot = (e[:, None] == eids[None, :]).astype(jnp.int32)
    occ = jnp.cumsum(onehot, axis=0)
    rank = jnp.sum(onehot * occ, axis=1) - 1
    counts = occ[-1]
    aligned = ((counts + BT - 1) // BT) * BT
    ends = jnp.cumsum(aligned)
    starts = ends - aligned
    total_used = ends[-1]

    slot_of_token = jnp.sum(onehot * starts[None, :], axis=1) + rank
    tok_ids = jnp.arange(T, dtype=jnp.int32)
    spread = jnp.arange(t_pad, dtype=jnp.int32) % T
    gather_idx = spread.at[slot_of_token].set(tok_ids)
    w_slot = jnp.zeros((t_pad, 1), jnp.float32).at[slot_of_token, 0].set(
        top_k_weights[:, 0])

    nb = t_pad // BT
    bstarts = jnp.arange(nb, dtype=jnp.int32) * BT
    owner = jnp.minimum(
        jnp.sum((ends[None, :] <= bstarts[:, None]).astype(jnp.int32), axis=1),
        num_experts - 1)
    valid = (bstarts < total_used).astype(jnp.int32)
    last_owner = jnp.minimum(
        jnp.sum((ends <= total_used - 1).astype(jnp.int32)),
        num_experts - 1)
    block_expert = jnp.where(valid == 1, owner, last_owner).astype(jnp.int32)
    return gather_idx, w_slot, slot_of_token, block_expert, valid


def kernel(hidden_states, top_k_index, top_k_weights, gate_up_proj, down_proj):
    T, H = hidden_states.shape
    E = gate_up_proj.shape[0]
    t_pad = T + E * BT

    gather_idx, w_slot, slot_of_token, block_expert, block_valid = (
        _routing_metadata(top_k_index, top_k_weights, E, t_pad))

    x_pad = _sc_row_gather(hidden_states, gather_idx, t_pad)
    y_pad = _grouped_mlp(x_pad, gate_up_proj, down_proj, w_slot,
                         block_expert, block_valid)
    return _sc_row_gather(y_pad, slot_of_token, T)
